# Initial kernel scaffold; baseline (speedup 1.0000x reference)
import jax
import jax.numpy as jnp
from jax.experimental import pallas as pl


def kernel(out, gts, size):
    raise NotImplementedError("write your pallas kernel here")


# speedup vs baseline: 3.7783x; 3.7783x over previous
import jax
import jax.numpy as jnp
import numpy as np
from jax.experimental import pallas as pl
from jax.experimental.pallas import tpu as pltpu

_OBJ_SCALE = 1.0
_NOOBJ_SCALE = 100.0
_IGNORE = 0.5
_ANCH = np.array([[10.0, 13.0], [16.0, 30.0], [33.0, 23.0]], dtype=np.float32)
_NA, _NB, _NH, _NW, _NGT = 3, 16, 64, 64, 256
_C5 = 85
_EPS = 1e-12
_TOTAL_CELLS = float(_NB * _NA * _NH * _NW)


def _dot(a, b):
    return jax.lax.dot_general(
        a, b, (((1,), (0,)), ((), ())),
        precision=jax.lax.Precision.HIGHEST,
        preferred_element_type=jnp.float32)


def _body(gr_ref, gc_ref, anch_ref, blk_ref, out_ref, acc_ref, dens_ref):
    b = pl.program_id(0)
    a = pl.program_id(1)

    @pl.when(jnp.logical_and(b == 0, a == 0))
    def _init():
        acc_ref[...] = jnp.zeros_like(acc_ref)
        dens_ref[0, 0] = 0.0

    bf = b.astype(jnp.float32)
    bat_c = gc_ref[:, 0:1]
    gx_c = gc_ref[:, 1:2] * _NW
    gy_c = gc_ref[:, 2:3] * _NH
    gi_c = jnp.floor(gx_c)
    gj_c = jnp.floor(gy_c)
    mine = bat_c == bf
    jio = jax.lax.broadcasted_iota(jnp.int32, (_NGT, _NH), 1)
    ohj = jnp.where((jio == gj_c.astype(jnp.int32)) & mine, 1.0, 0.0)
    iio = jax.lax.broadcasted_iota(jnp.int32, (_NGT, _NW), 1)
    ohi = jnp.where(iio == gi_c.astype(jnp.int32), 1.0, 0.0)

    vals = []
    for c in range(5):
        rows_c = _dot(ohj, blk_ref[0, 0, c])
        vals.append(jnp.sum(rows_c * ohi, axis=1, keepdims=True))
    v5 = jnp.concatenate(vals, axis=1)
    cat = jnp.concatenate([v5, v5, v5, jnp.zeros((_NGT, 1), jnp.float32)], axis=1)
    group = jax.lax.broadcasted_iota(jnp.int32, (_NGT, 16), 1) // 5
    acc_ref[...] += jnp.where(group == a, cat, 0.0)

    p4 = blk_ref[0, 0, 4]
    conf = jax.nn.sigmoid(p4)
    dens_ref[0, 0] += jnp.sum(-jnp.log(1.0 - conf + _EPS))

    @pl.when(jnp.logical_and(b == _NB - 1, a == _NA - 1))
    def _combine():
        gw_r = gr_ref[3:4, :] * _NW
        gh_r = gr_ref[4:5, :] * _NH
        gw_c = gc_ref[:, 3:4] * _NW
        gh_c = gc_ref[:, 4:5] * _NH

        def iou_ab(gw, gh, k):
            aw = anch_ref[0, k]
            ah = anch_ref[1, k]
            inter = jnp.minimum(gw, aw) * jnp.minimum(gh, ah)
            union = gw * gh + aw * ah - inter
            return inter / (union + 1e-16)

        iou_r = [iou_ab(gw_r, gh_r, k) for k in range(_NA)]
        iou_c = [iou_ab(gw_c, gh_c, k) for k in range(_NA)]

        def argmax3(i0, i1, i2):
            best = jnp.zeros_like(i0, dtype=jnp.int32)
            m = i0
            best = jnp.where(i1 > m, 1, best)
            m = jnp.maximum(m, i1)
            best = jnp.where(i2 > m, 2, best)
            return best

        best_r = argmax3(*iou_r)
        best_c = argmax3(*iou_c)
        excl_r = [(iou_r[k] > _IGNORE) | (best_r == k) for k in range(_NA)]
        excl_c = [(iou_c[k] > _IGNORE) | (best_c == k) for k in range(_NA)]

        bat_r = gr_ref[0:1, :]
        gx_r = gr_ref[1:2, :] * _NW
        gy_r = gr_ref[2:3, :] * _NH
        cell_r = (bat_r * _NH + jnp.floor(gy_r)) * _NW + jnp.floor(gx_r)
        cell_c = (gc_ref[:, 0:1] * _NH + gj_c) * _NW + gi_c

        io_r = jax.lax.broadcasted_iota(jnp.int32, (_NGT, _NGT), 1)
        io_c = jax.lax.broadcasted_iota(jnp.int32, (_NGT, _NGT), 0)
        later = io_r > io_c
        same_cell = cell_r == cell_c
        same_best = best_r == best_c
        winner_obj = ~jnp.any(same_cell & same_best & later, axis=1, keepdims=True)
        winner_excl = [
            excl_c[k] & ~jnp.any(same_cell & excl_r[k] & later, axis=1, keepdims=True)
            for k in range(_NA)]

        fobj = winner_obj.astype(jnp.float32)
        n_obj = jnp.maximum(jnp.sum(fobj), 1.0)
        n_excl = sum(jnp.sum(w.astype(jnp.float32)) for w in winner_excl)
        n_noobj = jnp.maximum(_TOTAL_CELLS - n_excl, 1.0)

        sel = [(best_c == k).astype(jnp.float32) for k in range(_NA)]
        pv = []
        for c in range(5):
            pv.append(sum(sel[k] * acc_ref[:, 5 * k + c:5 * k + c + 1]
                          for k in range(_NA)))
        conf_a = [acc_ref[:, 5 * k + 4:5 * k + 5] for k in range(_NA)]

        saw_sel = sum(sel[k] * anch_ref[0, k] for k in range(_NA))
        sah_sel = sum(sel[k] * anch_ref[1, k] for k in range(_NA))
        tx = gx_c - gi_c
        ty = gy_c - gj_c
        tw = gw_c / saw_sel
        th = gh_c / sah_sel

        xs = jax.nn.sigmoid(pv[0])
        ys = jax.nn.sigmoid(pv[1])
        bbox = (xs - tx) ** 2 + (ys - ty) ** 2 \
            + (pv[2] - jnp.log(tw)) ** 2 + (pv[3] - jnp.log(th)) ** 2
        obj_bce = -jnp.log(jax.nn.sigmoid(pv[4]) + _EPS)
        sum_bbox = jnp.sum(bbox * fobj)
        sum_objbce = jnp.sum(obj_bce * fobj)
        corr = sum(
            jnp.sum(jnp.where(winner_excl[k],
                              -jnp.log(1.0 - jax.nn.sigmoid(conf_a[k]) + _EPS),
                              0.0))
            for k in range(_NA))

        total = (sum_bbox + _OBJ_SCALE * sum_objbce) / n_obj \
            + _NOOBJ_SCALE * (dens_ref[0, 0] - corr) / n_noobj
        out_ref[0, 0] = total


def kernel(out, gts, size):
    out5 = out.reshape(_NB, _NA, _C5, _NH, _NW)
    stride_h = (size[0] // _NH).astype(jnp.float32)
    stride_w = (size[1] // _NW).astype(jnp.float32)
    saw = jnp.asarray(_ANCH[:, 0]) / stride_w
    sah = jnp.asarray(_ANCH[:, 1]) / stride_h
    anch = jnp.stack([saw, sah])
    gts_r = gts.T

    total = pl.pallas_call(
        _body,
        grid=(_NB, _NA),
        in_specs=[
            pl.BlockSpec((5, _NGT), lambda b, a: (0, 0)),
            pl.BlockSpec((_NGT, 5), lambda b, a: (0, 0)),
            pl.BlockSpec(memory_space=pltpu.SMEM),
            pl.BlockSpec((1, 1, 5, _NH, _NW), lambda b, a: (b, a, 0, 0, 0)),
        ],
        out_specs=pl.BlockSpec(memory_space=pltpu.SMEM),
        out_shape=jax.ShapeDtypeStruct((1, 1), jnp.float32),
        scratch_shapes=[
            pltpu.VMEM((_NGT, 16), jnp.float32),
            pltpu.SMEM((1, 1), jnp.float32),
        ],
    )(gts_r, gts, anch, out5)
    return total.reshape(())


# speedup vs baseline: 5.5511x; 1.4692x over previous
import jax
import jax.numpy as jnp
import numpy as np
from jax.experimental import pallas as pl
from jax.experimental.pallas import tpu as pltpu

_OBJ_SCALE = 1.0
_NOOBJ_SCALE = 100.0
_IGNORE = 0.5
_ANCH = np.array([[10.0, 13.0], [16.0, 30.0], [33.0, 23.0]], dtype=np.float32)
_NA, _NB, _NH, _NW, _NGT = 3, 16, 64, 64, 256
_C5 = 85
_EPS = 1e-12
_TOTAL_CELLS = float(_NB * _NA * _NH * _NW)


def _body(gr_ref, gc_ref, anch_ref, blk0, blk1, blk2, out_ref, acc_ref, dens_ref):
    b = pl.program_id(0)

    @pl.when(b == 0)
    def _init():
        acc_ref[...] = jnp.zeros_like(acc_ref)
        dens_ref[0, 0] = 0.0

    bf = b.astype(jnp.float32)
    bat_r = gr_ref[0:1, :]
    gx_r = gr_ref[1:2, :] * _NW
    gy_r = gr_ref[2:3, :] * _NH
    gi_r = jnp.floor(gx_r).astype(jnp.int32)
    gj_r = jnp.floor(gy_r).astype(jnp.int32)
    mine = bat_r == bf

    iio = jax.lax.broadcasted_iota(jnp.int32, (_NW, _NGT), 0)
    ohiT = jnp.where((iio == gi_r) & mine, 1.0, 0.0)
    jio = jax.lax.broadcasted_iota(jnp.int32, (_NH, _NGT), 0)
    ohjT = jnp.where(jio == gj_r, 1.0, 0.0)

    dens = 0.0
    rows = []
    for a, blk in enumerate((blk0, blk1, blk2)):
        p = blk[0]
        tmp = jax.lax.dot_general(
            p, ohiT, (((1,), (0,)), ((), ())),
            preferred_element_type=jnp.float32)
        for c in range(5):
            sl = tmp[c * _NH:(c + 1) * _NH, :]
            rows.append(jnp.sum(sl * ohjT, axis=0, keepdims=True))
        conf = jax.nn.sigmoid(p[4 * _NH:5 * _NH, :])
        dens = dens + jnp.sum(-jnp.log(1.0 - conf + _EPS))
    acc_ref[0:15, :] += jnp.concatenate(rows, axis=0)
    dens_ref[0, 0] += dens

    @pl.when(b == _NB - 1)
    def _combine():
        gw_r = gr_ref[3:4, :] * _NW
        gh_r = gr_ref[4:5, :] * _NH
        gw_c = gc_ref[:, 3:4] * _NW
        gh_c = gc_ref[:, 4:5] * _NH

        def iou_ab(gw, gh, k):
            aw = anch_ref[0, k]
            ah = anch_ref[1, k]
            inter = jnp.minimum(gw, aw) * jnp.minimum(gh, ah)
            union = gw * gh + aw * ah - inter
            return inter / (union + 1e-16)

        iou_r = [iou_ab(gw_r, gh_r, k) for k in range(_NA)]
        iou_c = [iou_ab(gw_c, gh_c, k) for k in range(_NA)]

        def argmax3(i0, i1, i2):
            best = jnp.zeros_like(i0, dtype=jnp.int32)
            m = i0
            best = jnp.where(i1 > m, 1, best)
            m = jnp.maximum(m, i1)
            best = jnp.where(i2 > m, 2, best)
            return best

        best_r = argmax3(*iou_r)
        best_c = argmax3(*iou_c)
        excl_r = [(iou_r[k] > _IGNORE) | (best_r == k) for k in range(_NA)]
        excl_c = [(iou_c[k] > _IGNORE) | (best_c == k) for k in range(_NA)]

        gx_c = gc_ref[:, 1:2] * _NW
        gy_c = gc_ref[:, 2:3] * _NH
        cell_r = (gr_ref[0:1, :] * _NH + jnp.floor(gy_r)) * _NW + jnp.floor(gx_r)
        cell_c = (gc_ref[:, 0:1] * _NH + jnp.floor(gy_c)) * _NW + jnp.floor(gx_c)

        io_g = jax.lax.broadcasted_iota(jnp.int32, (_NGT, _NGT), 1)
        io_gp = jax.lax.broadcasted_iota(jnp.int32, (_NGT, _NGT), 0)
        later = io_gp > io_g
        same_cell = (cell_r == cell_c) & later
        same_best = best_r == best_c
        winner_obj = ~jnp.any(same_cell & same_best, axis=0, keepdims=True)
        winner_excl = [
            excl_r[k] & ~jnp.any(same_cell & excl_c[k], axis=0, keepdims=True)
            for k in range(_NA)]

        fobj = winner_obj.astype(jnp.float32)
        n_obj = jnp.maximum(jnp.sum(fobj), 1.0)
        n_excl = sum(jnp.sum(w.astype(jnp.float32)) for w in winner_excl)
        n_noobj = jnp.maximum(_TOTAL_CELLS - n_excl, 1.0)

        sel = [(best_r == k).astype(jnp.float32) for k in range(_NA)]
        pv = []
        for c in range(5):
            pv.append(sum(sel[k] * acc_ref[5 * k + c:5 * k + c + 1, :]
                          for k in range(_NA)))
        conf_a = [acc_ref[5 * k + 4:5 * k + 5, :] for k in range(_NA)]

        saw_sel = sum(sel[k] * anch_ref[0, k] for k in range(_NA))
        sah_sel = sum(sel[k] * anch_ref[1, k] for k in range(_NA))
        tx = gx_r - jnp.floor(gx_r)
        ty = gy_r - jnp.floor(gy_r)
        tw = gw_r / saw_sel
        th = gh_r / sah_sel

        xs = jax.nn.sigmoid(pv[0])
        ys = jax.nn.sigmoid(pv[1])
        bbox = (xs - tx) ** 2 + (ys - ty) ** 2 \
            + (pv[2] - jnp.log(tw)) ** 2 + (pv[3] - jnp.log(th)) ** 2
        obj_bce = -jnp.log(jax.nn.sigmoid(pv[4]) + _EPS)
        sum_bbox = jnp.sum(bbox * fobj)
        sum_objbce = jnp.sum(obj_bce * fobj)
        corr = sum(
            jnp.sum(jnp.where(winner_excl[k],
                              -jnp.log(1.0 - jax.nn.sigmoid(conf_a[k]) + _EPS),
                              0.0))
            for k in range(_NA))

        total = (sum_bbox + _OBJ_SCALE * sum_objbce) / n_obj \
            + _NOOBJ_SCALE * (dens_ref[0, 0] - corr) / n_noobj
        out_ref[0, 0] = total


def kernel(out, gts, size):
    out_rs = out.reshape(_NB, 255 * _NH, _NW)
    stride_h = (size[0] // _NH).astype(jnp.float32)
    stride_w = (size[1] // _NW).astype(jnp.float32)
    saw = jnp.asarray(_ANCH[:, 0]) / stride_w
    sah = jnp.asarray(_ANCH[:, 1]) / stride_h
    anch = jnp.stack([saw, sah])
    gts_r = gts.T

    blk = 5 * _NH
    total = pl.pallas_call(
        _body,
        grid=(_NB,),
        in_specs=[
            pl.BlockSpec((5, _NGT), lambda b: (0, 0)),
            pl.BlockSpec((_NGT, 5), lambda b: (0, 0)),
            pl.BlockSpec(memory_space=pltpu.SMEM),
            pl.BlockSpec((1, blk, _NW), lambda b: (b, 0, 0)),
            pl.BlockSpec((1, blk, _NW), lambda b: (b, 17, 0)),
            pl.BlockSpec((1, blk, _NW), lambda b: (b, 34, 0)),
        ],
        out_specs=pl.BlockSpec(memory_space=pltpu.SMEM),
        out_shape=jax.ShapeDtypeStruct((1, 1), jnp.float32),
        scratch_shapes=[
            pltpu.VMEM((16, _NGT), jnp.float32),
            pltpu.SMEM((1, 1), jnp.float32),
        ],
    )(gts_r, gts, anch, out_rs, out_rs, out_rs)
    return total.reshape(())
